# baseline (device time: 27790 ns/iter reference)
import jax
import jax.numpy as jnp
from jax import lax
from jax.experimental import pallas as pl
from jax.experimental.pallas import tpu as pltpu

N_DEV = 32
NP = 4
NQ = 8


def kernel(x, w_mat):
    m_per, k = x.shape
    n = w_mat.shape[1]
    n_per = n // N_DEV
    rows_per_plane = NQ * m_per

    def body(x_ref, w_ref, out_ref, yblk_ref, stage1_ref, s2_ref,
             s1_sems, r1_sems, s2_sems, r2_sems):
        me = lax.axis_index("i")
        p = me // NQ
        q = lax.rem(me, NQ)

        bs = pltpu.get_barrier_semaphore()
        for t in range(1, N_DEV):
            pl.semaphore_signal(bs, inc=1,
                                device_id=(lax.rem(me + t, N_DEV),),
                                device_id_type=pl.DeviceIdType.MESH)
        pl.semaphore_wait(bs, N_DEV - 1)

        xv = x_ref[:, :]

        s1_descr = []
        for qq in range(NQ // 2):
            for p4 in range(NP):
                g = p4 * (NQ // 2) + qq
                t = jnp.dot(xv, w_ref[:, g * 128:(g + 1) * 128],
                            preferred_element_type=jnp.float32
                            ).astype(jnp.bfloat16)
                yblk_ref[(2 * qq) * NP + p4, :, :] = t[:, :n_per]
                yblk_ref[(2 * qq + 1) * NP + p4, :, :] = t[:, n_per:]
            for dq in (2 * qq, 2 * qq + 1):
                rdma = pltpu.make_async_remote_copy(
                    src_ref=yblk_ref.at[pl.ds(dq * NP, NP)],
                    dst_ref=stage1_ref.at[q],
                    send_sem=s1_sems.at[dq],
                    recv_sem=r1_sems.at[q],
                    device_id=(p * NQ + dq,),
                    device_id_type=pl.DeviceIdType.MESH,
                )
                s1_descr.append((dq, rdma))

                @pl.when(dq != q)
                def _(rdma=rdma):
                    rdma.start()

        for p4 in range(NP):
            stage1_ref[q, p4, :, :] = yblk_ref[q * NP + p4, :, :]

        for sl in range(NQ):
            recv = pltpu.make_async_remote_copy(
                src_ref=yblk_ref.at[pl.ds(0, NP)],
                dst_ref=stage1_ref.at[sl],
                send_sem=s1_sems.at[0],
                recv_sem=r1_sems.at[sl],
                device_id=(me,),
                device_id_type=pl.DeviceIdType.MESH,
            )

            @pl.when(sl != q)
            def _(recv=recv):
                recv.wait_recv()

        for p4 in range(NP):
            for qs in range(NQ):
                s2_ref[p4, pl.ds(qs * m_per, m_per), :] = stage1_ref[qs, p4, :, :]

        s2_descr = []
        for p4 in range(NP):
            rdma = pltpu.make_async_remote_copy(
                src_ref=s2_ref.at[p4],
                dst_ref=out_ref.at[pl.ds(p * rows_per_plane, rows_per_plane), :],
                send_sem=s2_sems.at[p4],
                recv_sem=r2_sems.at[p],
                device_id=(p4 * NQ + q,),
                device_id_type=pl.DeviceIdType.MESH,
            )
            s2_descr.append((p4, rdma))

            @pl.when(p4 != p)
            def _(rdma=rdma):
                rdma.start()

        out_ref[pl.ds(p * rows_per_plane, rows_per_plane), :] = s2_ref[p, :, :]

        for pp in range(NP):
            recv = pltpu.make_async_remote_copy(
                src_ref=s2_ref.at[0],
                dst_ref=out_ref.at[pl.ds(pp * rows_per_plane, rows_per_plane), :],
                send_sem=s2_sems.at[0],
                recv_sem=r2_sems.at[pp],
                device_id=(me,),
                device_id_type=pl.DeviceIdType.MESH,
            )

            @pl.when(pp != p)
            def _(recv=recv):
                recv.wait_recv()

        for dq, rdma in s1_descr:
            @pl.when(dq != q)
            def _(rdma=rdma):
                rdma.wait_send()
        for p4, rdma in s2_descr:
            @pl.when(p4 != p)
            def _(rdma=rdma):
                rdma.wait_send()

    out_shape = jax.ShapeDtypeStruct((N_DEV * m_per, n_per), jnp.bfloat16)
    return pl.pallas_call(
        body,
        out_shape=out_shape,
        in_specs=[
            pl.BlockSpec(memory_space=pltpu.VMEM),
            pl.BlockSpec(memory_space=pltpu.VMEM),
        ],
        out_specs=pl.BlockSpec(memory_space=pltpu.VMEM),
        scratch_shapes=[
            pltpu.VMEM((N_DEV, m_per, n_per), jnp.bfloat16),
            pltpu.VMEM((NQ, NP, m_per, n_per), jnp.bfloat16),
            pltpu.VMEM((NP, NQ * m_per, n_per), jnp.bfloat16),
            pltpu.SemaphoreType.DMA((NQ,)),
            pltpu.SemaphoreType.DMA((NQ,)),
            pltpu.SemaphoreType.DMA((NP,)),
            pltpu.SemaphoreType.DMA((NP,)),
        ],
        compiler_params=pltpu.CompilerParams(collective_id=0),
    )(x, w_mat)


# device time: 25879 ns/iter; 1.0738x vs baseline; 1.0738x over previous
import jax
import jax.numpy as jnp
from jax import lax
from jax.experimental import pallas as pl
from jax.experimental.pallas import tpu as pltpu

N_DEV = 32
NP = 4
NQ = 8


def kernel(x, w_mat):
    m_per, k = x.shape
    n = w_mat.shape[1]
    n_per = n // N_DEV
    rows_per_plane = NQ * m_per

    def body(x_ref, w_ref, out_ref, yblk_ref, stage1_ref, s2_ref,
             s1_sems, r1_sems, s2_sems, r2_sems):
        me = lax.axis_index("i")
        p = me // NQ
        q = lax.rem(me, NQ)

        bs = pltpu.get_barrier_semaphore()
        for t in range(1, NQ):
            pl.semaphore_signal(bs, inc=1,
                                device_id=(p * NQ + lax.rem(q + t, NQ),),
                                device_id_type=pl.DeviceIdType.MESH)
        for t in range(1, NP):
            pl.semaphore_signal(bs, inc=8,
                                device_id=(lax.rem(p + t, NP) * NQ + q,),
                                device_id_type=pl.DeviceIdType.MESH)

        xv = x_ref[:, :]

        s1_descr = []
        for qq in range(NQ // 2):
            for p4 in range(NP):
                g = p4 * (NQ // 2) + qq
                t = jnp.dot(xv, w_ref[:, g * 128:(g + 1) * 128],
                            preferred_element_type=jnp.float32
                            ).astype(jnp.bfloat16)
                yblk_ref[(2 * qq) * NP + p4, :, :] = t[:, :n_per]
                yblk_ref[(2 * qq + 1) * NP + p4, :, :] = t[:, n_per:]
            if qq == 0:
                pl.semaphore_wait(bs, (NQ - 1) + (NP - 1) * 8)
            for dq in (2 * qq, 2 * qq + 1):
                rdma = pltpu.make_async_remote_copy(
                    src_ref=yblk_ref.at[pl.ds(dq * NP, NP)],
                    dst_ref=stage1_ref.at[q],
                    send_sem=s1_sems.at[dq],
                    recv_sem=r1_sems.at[q],
                    device_id=(p * NQ + dq,),
                    device_id_type=pl.DeviceIdType.MESH,
                )
                s1_descr.append((dq, rdma))

                @pl.when(dq != q)
                def _(rdma=rdma):
                    rdma.start()

        for p4 in range(NP):
            stage1_ref[q, p4, :, :] = yblk_ref[q * NP + p4, :, :]

        for sl in range(NQ):
            recv = pltpu.make_async_remote_copy(
                src_ref=yblk_ref.at[pl.ds(0, NP)],
                dst_ref=stage1_ref.at[sl],
                send_sem=s1_sems.at[0],
                recv_sem=r1_sems.at[sl],
                device_id=(me,),
                device_id_type=pl.DeviceIdType.MESH,
            )

            @pl.when(sl != q)
            def _(recv=recv):
                recv.wait_recv()

        for p4 in range(NP):
            for qs in range(NQ):
                s2_ref[p4, pl.ds(qs * m_per, m_per), :] = stage1_ref[qs, p4, :, :]

        s2_descr = []
        for p4 in range(NP):
            rdma = pltpu.make_async_remote_copy(
                src_ref=s2_ref.at[p4],
                dst_ref=out_ref.at[pl.ds(p * rows_per_plane, rows_per_plane), :],
                send_sem=s2_sems.at[p4],
                recv_sem=r2_sems.at[p],
                device_id=(p4 * NQ + q,),
                device_id_type=pl.DeviceIdType.MESH,
            )
            s2_descr.append((p4, rdma))

            @pl.when(p4 != p)
            def _(rdma=rdma):
                rdma.start()

        out_ref[pl.ds(p * rows_per_plane, rows_per_plane), :] = s2_ref[p, :, :]

        for pp in range(NP):
            recv = pltpu.make_async_remote_copy(
                src_ref=s2_ref.at[0],
                dst_ref=out_ref.at[pl.ds(pp * rows_per_plane, rows_per_plane), :],
                send_sem=s2_sems.at[0],
                recv_sem=r2_sems.at[pp],
                device_id=(me,),
                device_id_type=pl.DeviceIdType.MESH,
            )

            @pl.when(pp != p)
            def _(recv=recv):
                recv.wait_recv()

        for dq, rdma in s1_descr:
            @pl.when(dq != q)
            def _(rdma=rdma):
                rdma.wait_send()
        for p4, rdma in s2_descr:
            @pl.when(p4 != p)
            def _(rdma=rdma):
                rdma.wait_send()

    out_shape = jax.ShapeDtypeStruct((N_DEV * m_per, n_per), jnp.bfloat16)
    return pl.pallas_call(
        body,
        out_shape=out_shape,
        in_specs=[
            pl.BlockSpec(memory_space=pltpu.VMEM),
            pl.BlockSpec(memory_space=pltpu.VMEM),
        ],
        out_specs=pl.BlockSpec(memory_space=pltpu.VMEM),
        scratch_shapes=[
            pltpu.VMEM((N_DEV, m_per, n_per), jnp.bfloat16),
            pltpu.VMEM((NQ, NP, m_per, n_per), jnp.bfloat16),
            pltpu.VMEM((NP, NQ * m_per, n_per), jnp.bfloat16),
            pltpu.SemaphoreType.DMA((NQ,)),
            pltpu.SemaphoreType.DMA((NQ,)),
            pltpu.SemaphoreType.DMA((NP,)),
            pltpu.SemaphoreType.DMA((NP,)),
        ],
        compiler_params=pltpu.CompilerParams(collective_id=0),
    )(x, w_mat)
